# baseline (speedup 1.0000x reference)
import jax
import jax.numpy as jnp
from jax.experimental import pallas as pl
from jax.experimental.pallas import tpu as pltpu

D_MODEL = 4096
NUM_EXPERTS = 64
TOP_K = 8
BLOCK_T = 512


def _gate_body_t(x_ref, wt_ref, bt_ref, vals_ref, idx_ref):
    logits_t = jax.lax.dot_general(
        wt_ref[...], x_ref[...],
        dimension_numbers=(((1,), (1,)), ((), ())),
        preferred_element_type=jnp.float32,
    ) + bt_ref[...]
    m = jnp.max(logits_t, axis=0, keepdims=True)
    e = jnp.exp(logits_t - m)
    denom = jnp.sum(e, axis=0, keepdims=True)

    bits = jax.lax.bitcast_convert_type(e, jnp.int32)
    iota = jax.lax.broadcasted_iota(jnp.int32, e.shape, 0)
    key = (bits & ~0x3F) | (NUM_EXPERTS - 1 - iota)
    keys = []
    work = key
    for _ in range(TOP_K):
        mx = jnp.max(work, axis=0, keepdims=True)
        keys.append(mx)
        work = jnp.where(work == mx, -1, work)
    top = jnp.concatenate(keys, axis=0)
    idx_ref[...] = (NUM_EXPERTS - 1) - (top & 0x3F)
    vals_ref[...] = (
        jax.lax.bitcast_convert_type(top & ~0x3F, jnp.float32) / denom
    )


@jax.jit
def kernel(x, W_gate, b_gate):
    n_tokens = x.shape[0]
    grid = (n_tokens // BLOCK_T,)
    wt = W_gate.T
    bt = b_gate.reshape(NUM_EXPERTS, 1)
    vals_t, idx_t = pl.pallas_call(
        _gate_body_t,
        grid=grid,
        in_specs=[
            pl.BlockSpec((BLOCK_T, D_MODEL), lambda i: (i, 0)),
            pl.BlockSpec((NUM_EXPERTS, D_MODEL), lambda i: (0, 0)),
            pl.BlockSpec((NUM_EXPERTS, 1), lambda i: (0, 0)),
        ],
        out_specs=[
            pl.BlockSpec((TOP_K, BLOCK_T), lambda i: (0, i)),
            pl.BlockSpec((TOP_K, BLOCK_T), lambda i: (0, i)),
        ],
        out_shape=[
            jax.ShapeDtypeStruct((TOP_K, n_tokens), jnp.float32),
            jax.ShapeDtypeStruct((TOP_K, n_tokens), jnp.int32),
        ],
        compiler_params=pltpu.CompilerParams(
            dimension_semantics=("parallel",),
        ),
    )(x, wt, bt)
    return vals_t.T, idx_t.T
